# Initial kernel scaffold; baseline (speedup 1.0000x reference)
import jax
import jax.numpy as jnp
from jax.experimental import pallas as pl

_V = 32000
_MASK_ID = 31999
_P_UNIFORM = 0.1
_U = _P_UNIFORM / _V

_BS = 128


def _gidd_block(z_ref, coef_ref, x_ref, o_ref):
    x = x_ref[...]
    bs, v = x.shape
    v_idx = jax.lax.broadcasted_iota(jnp.int32, (bs, v), 1)
    mask_col = v_idx == _MASK_ID
    x = jnp.where(mask_col, -1e6, x)

    m = jnp.max(x, axis=1, keepdims=True)
    z = z_ref[...]
    onehot = v_idx == z
    x_z = jnp.sum(jnp.where(onehot, x, 0.0), axis=1, keepdims=True)
    e = jnp.exp(x - m)
    zsum = jnp.sum(e, axis=1, keepdims=True)

    coef = coef_ref[...]
    t = coef[:, 0:1]
    a_t = coef[:, 1:2]
    s = coef[:, 2:3]
    a_s = coef[:, 3:4]
    a_ts = coef[:, 4:5]
    c_ts = coef[:, 5:6]

    mask_hit = (z == _MASK_ID).astype(x.dtype)
    pi_z = _U + 0.9 * mask_hit
    p_z = jnp.exp(x_z - m) / zsum
    q_zt = a_t * p_z + t * pi_z
    g0 = (pi_z * c_ts) / q_zt
    g1 = a_ts / q_zt

    q_s = (a_s / zsum) * e + (s * _U)
    q_s = jnp.where(mask_col, q_s + s * 0.9, q_s)
    o_ref[...] = jnp.where(onehot, g0 + g1, g0) * q_s


def kernel(logits, z_t, t, s):
    B, S, V = logits.shape
    R = B * S
    x = logits.reshape(R, V)
    z = z_t.reshape(R, 1).astype(jnp.int32)

    a_t = 1.0 - t
    a_s = 1.0 - s
    a_ts = a_t / a_s
    c_ts = t - a_ts * s
    zero = jnp.zeros_like(t)
    coef_b = jnp.stack([t, a_t, s, a_s, a_ts, c_ts, zero, zero], axis=1)
    coef = jnp.broadcast_to(coef_b[:, None, :], (B, S, 8)).reshape(R, 8)

    out = pl.pallas_call(
        _gidd_block,
        grid=(R // _BS,),
        in_specs=[
            pl.BlockSpec((_BS, 1), lambda i: (i, 0)),
            pl.BlockSpec((_BS, 8), lambda i: (i, 0)),
            pl.BlockSpec((_BS, V), lambda i: (i, 0)),
        ],
        out_specs=pl.BlockSpec((_BS, V), lambda i: (i, 0)),
        out_shape=jax.ShapeDtypeStruct((R, V), jnp.float32),
    )(z, coef, x)
    return out.reshape(B, S, V)


# speedup vs baseline: 3.7871x; 3.7871x over previous
import jax
import jax.numpy as jnp
from jax.experimental import pallas as pl

_V = 32000
_MASK_ID = 31999
_P_UNIFORM = 0.1
_U = _P_UNIFORM / _V

_BS = 64


def _gidd_block(z_ref, coef_ref, x_ref, o_ref):
    x = x_ref[...]
    bs, v = x.shape
    v_idx = jax.lax.broadcasted_iota(jnp.int32, (bs, v), 1)
    mask_col = v_idx == _MASK_ID
    x = jnp.where(mask_col, -1e6, x)

    m = jnp.max(x, axis=1, keepdims=True)
    z = z_ref[...]
    onehot = v_idx == z
    e = jnp.exp(x - m)
    zsum = jnp.sum(e, axis=1, keepdims=True)
    e_z = jnp.sum(jnp.where(onehot, e, 0.0), axis=1, keepdims=True)

    coef = coef_ref[...]
    t = coef[:, 0:1]
    a_t = coef[:, 1:2]
    s = coef[:, 2:3]
    a_s = coef[:, 3:4]
    a_ts = coef[:, 4:5]
    c_ts = coef[:, 5:6]

    mask_hit = (z == _MASK_ID).astype(e.dtype)
    pi_z = _U + 0.9 * mask_hit
    p_z = e_z / zsum
    q_zt = a_t * p_z + t * pi_z
    g0 = (pi_z * c_ts) / q_zt
    g1 = a_ts / q_zt

    q_s = (a_s / zsum) * e + (s * _U)
    q_s = jnp.where(mask_col, q_s + s * 0.9, q_s)
    o_ref[...] = jnp.where(onehot, g0 + g1, g0) * q_s


def kernel(logits, z_t, t, s):
    B, S, V = logits.shape
    R = B * S
    x = logits.reshape(R, V)
    z = z_t.reshape(R, 1).astype(jnp.int32)

    a_t = 1.0 - t
    a_s = 1.0 - s
    a_ts = a_t / a_s
    c_ts = t - a_ts * s
    zero = jnp.zeros_like(t)
    coef_b = jnp.stack([t, a_t, s, a_s, a_ts, c_ts, zero, zero], axis=1)
    coef = jnp.broadcast_to(coef_b[:, None, :], (B, S, 8)).reshape(R, 8)

    out = pl.pallas_call(
        _gidd_block,
        grid=(R // _BS,),
        in_specs=[
            pl.BlockSpec((_BS, 1), lambda i: (i, 0)),
            pl.BlockSpec((_BS, 8), lambda i: (i, 0)),
            pl.BlockSpec((_BS, V), lambda i: (i, 0)),
        ],
        out_specs=pl.BlockSpec((_BS, V), lambda i: (i, 0)),
        out_shape=jax.ShapeDtypeStruct((R, V), jnp.float32),
    )(z, coef, x)
    return out.reshape(B, S, V)
